# baseline (speedup 1.0000x reference)
import functools

import jax
import jax.numpy as jnp
from jax import lax
from jax.experimental import pallas as pl
from jax.experimental.pallas import tpu as pltpu
from jax.experimental.pallas import tpu_sc as plsc

VOCAB = 1000000
DIM = 64
REG = 1e-06
N_NEG = 20

NC = 2
NS = 16
NW = NC * NS
L = 16

CB = 16
NEG_ROWS = CB * N_NEG


def _sc_gather_dot(U2, V2, P2, up2, uof, vp2, vof, vn2, vnof, B):
    nb = B // NW
    nch = nb // CB
    nneg = nb * N_NEG

    mesh = plsc.VectorSubcoreMesh(core_axis_name="c", subcore_axis_name="s")

    @functools.partial(
        pl.kernel,
        out_type=(
            jax.ShapeDtypeStruct((B, L), jnp.float32),
            jax.ShapeDtypeStruct((B, L), jnp.float32),
            jax.ShapeDtypeStruct((NW, L), jnp.float32),
        ),
        mesh=mesh,
        scratch_types=[
            pltpu.VMEM((nb,), jnp.int32),
            pltpu.VMEM((nb + L,), jnp.int32),
            pltpu.VMEM((nb,), jnp.int32),
            pltpu.VMEM((nb + L,), jnp.int32),
            pltpu.VMEM((nneg,), jnp.int32),
            pltpu.VMEM((nneg + 2 * L,), jnp.int32),
            pltpu.VMEM((CB, 128), jnp.float32),
            pltpu.VMEM((CB, 128), jnp.float32),
            pltpu.VMEM((CB, 128), jnp.float32),
            pltpu.VMEM((NEG_ROWS, 128), jnp.float32),
            pltpu.VMEM((CB, L), jnp.float32),
            pltpu.VMEM((CB, L), jnp.float32),
            pltpu.VMEM((L,), jnp.float32),
            pltpu.SemaphoreType.DMA,
        ],
    )
    def k(u_hbm, v_hbm, p_hbm, up_hbm, uo_hbm, vp_hbm, vo_hbm, vn_hbm, no_hbm,
          pos_out, neg_out, reg_out,
          u2v, uov, v2v, vov, n2v, nov,
          urows, vrows, prows, nrows, posb, negb, regv, sem):
        wid = lax.axis_index("s") * NC + lax.axis_index("c")
        base = wid * nb
        zero = jnp.zeros((L,), jnp.float32)

        def c_body(ci, racc):
            b0 = base + ci * CB
            c0 = ci * CB
            n0 = ci * NEG_ROWS
            hs = [pltpu.async_copy(u_hbm.at[u2v.at[pl.ds(c0, CB)]], urows, sem),
                  pltpu.async_copy(v_hbm.at[v2v.at[pl.ds(c0, CB)]], vrows, sem),
                  pltpu.async_copy(p_hbm.at[u2v.at[pl.ds(c0, CB)]], prows, sem),
                  pltpu.async_copy(v_hbm.at[n2v.at[pl.ds(n0, 128)]],
                                   nrows.at[pl.ds(0, 128)], sem),
                  pltpu.async_copy(v_hbm.at[n2v.at[pl.ds(n0 + 128, 128)]],
                                   nrows.at[pl.ds(128, 128)], sem),
                  pltpu.async_copy(v_hbm.at[n2v.at[pl.ds(n0 + 256, 64)]],
                                   nrows.at[pl.ds(256, 64)], sem)]
            for h in hs:
                h.wait()

            def b_body(b, racc):
                bg = c0 + b
                pu = uov[pl.ds(bg, L)][0]
                pv = vov[pl.ds(bg, L)][0]
                no1 = nov[pl.ds(n0 + b * N_NEG, L)]
                no2 = nov[pl.ds(n0 + b * N_NEG + L, L)]
                u = [urows[b, pl.ds(pu + L * t, L)] for t in range(4)]
                v = [vrows[b, pl.ds(pv + L * t, L)] for t in range(4)]
                pp = [prows[b, pl.ds(pu + L * t, L)] for t in range(4)]
                posb[b, :] = u[0] * v[0] + u[1] * v[1] + u[2] * v[2] + u[3] * v[3]
                racc = (racc + jnp.abs(u[0] - pp[0]) + jnp.abs(u[1] - pp[1])
                        + jnp.abs(u[2] - pp[2]) + jnp.abs(u[3] - pp[3]))

                a = [zero, zero, zero, zero]
                for n in range(N_NEG):
                    po = no1[n] if n < L else no2[n - L]
                    r = b * N_NEG + n
                    for t in range(4):
                        a[t] = a[t] + nrows[r, pl.ds(po + L * t, L)]
                negb[b, :] = a[0] * u[0] + a[1] * u[1] + a[2] * u[2] + a[3] * u[3]
                return racc

            racc = lax.fori_loop(0, CB, b_body, racc)
            pltpu.sync_copy(posb, pos_out.at[pl.ds(b0, CB)])
            pltpu.sync_copy(negb, neg_out.at[pl.ds(b0, CB)])
            return racc

        pltpu.sync_copy(up_hbm.at[pl.ds(base, nb)], u2v)
        pltpu.sync_copy(uo_hbm.at[pl.ds(base, nb)], uov.at[pl.ds(0, nb)])
        pltpu.sync_copy(vp_hbm.at[pl.ds(base, nb)], v2v)
        pltpu.sync_copy(vo_hbm.at[pl.ds(base, nb)], vov.at[pl.ds(0, nb)])
        pltpu.sync_copy(vn_hbm.at[pl.ds(wid * nneg, nneg)], n2v)
        pltpu.sync_copy(no_hbm.at[pl.ds(wid * nneg, nneg)], nov.at[pl.ds(0, nneg)])
        racc = lax.fori_loop(0, nch, c_body, zero)
        regv[...] = racc
        pltpu.sync_copy(regv, reg_out.at[wid])

    return k(U2, V2, P2, up2, uof, vp2, vof, vn2, vnof)


def _tc_finalize(pos, neg, regp, B):
    def body(pos_ref, neg_ref, reg_ref, o_ref):
        s = jnp.sum(pos_ref[...], axis=1)
        t = jnp.sum(neg_ref[...], axis=1)
        ls = jnp.minimum(s, 0.0) - jnp.log1p(jnp.exp(-jnp.abs(s)))
        lt = jnp.minimum(-t, 0.0) - jnp.log1p(jnp.exp(-jnp.abs(t)))
        total = jnp.sum(ls + lt)
        reg = REG * jnp.sum(reg_ref[...])
        o_ref[...] = jnp.reshape(-(total / B) - reg, (1, 1))

    return pl.pallas_call(
        body, out_shape=jax.ShapeDtypeStruct((1, 1), jnp.float32),
    )(pos, neg, regp)


def kernel(U, V, pretrained, u_pos, v_pos, v_neg, batch_size):
    B = u_pos.shape[0]
    U2 = U.reshape(VOCAB // 2, 2 * DIM)
    V2 = V.reshape(VOCAB // 2, 2 * DIM)
    P2 = pretrained.reshape(VOCAB // 2, 2 * DIM)
    up = u_pos.astype(jnp.int32)
    vp = v_pos.astype(jnp.int32)
    vn = v_neg.astype(jnp.int32).reshape(B * N_NEG)
    up2, uof = up >> 1, (up & 1) << 6
    vp2, vof = vp >> 1, (vp & 1) << 6
    vn2, vnof = vn >> 1, (vn & 1) << 6
    pos, neg, regp = _sc_gather_dot(U2, V2, P2, up2, uof, vp2, vof, vn2, vnof, B)
    out = _tc_finalize(pos, neg, regp, B)
    return out[0, 0]


# speedup vs baseline: 1.1110x; 1.1110x over previous
import functools

import jax
import jax.numpy as jnp
from jax import lax
from jax.experimental import pallas as pl
from jax.experimental.pallas import tpu as pltpu
from jax.experimental.pallas import tpu_sc as plsc

VOCAB = 1000000
DIM = 64
REG = 1e-06
N_NEG = 20

NC = 2
NS = 16
NW = NC * NS
L = 16

CB = 32
NEG_ROWS = CB * N_NEG
NIDX_ROWS = NEG_ROWS // 128

_MESH = dict(core_axis_name="c", subcore_axis_name="s")


def _sc_uv_kernel(U, V, u_pos, v_pos, vneg2, B):
    nb = B // NW
    nch = nb // CB
    nidx_per_w = nb * N_NEG // 128

    @functools.partial(
        pl.kernel,
        out_type=(
            jax.ShapeDtypeStruct((B, L), jnp.float32),
            jax.ShapeDtypeStruct((B, L), jnp.float32),
        ),
        mesh=plsc.VectorSubcoreMesh(**_MESH),
        scratch_types=[
            pltpu.VMEM((nb,), jnp.int32),
            pltpu.VMEM((nb,), jnp.int32),
            pltpu.VMEM((nidx_per_w, 128), jnp.int32),
            pltpu.VMEM((CB, 2 * DIM), jnp.float32),
            pltpu.VMEM((CB, 2 * DIM), jnp.float32),
            pltpu.VMEM((NEG_ROWS, 2 * DIM), jnp.float32),
            pltpu.VMEM((CB, L), jnp.float32),
            pltpu.VMEM((CB, L), jnp.float32),
            pltpu.SemaphoreType.DMA,
        ],
    )
    def k(u_hbm, v_hbm, up_hbm, vp_hbm, vn_hbm,
          pos_out, neg_out,
          uidx, vidx, nidx, urows, vrows, nrows, posb, negb, sem):
        wid = lax.axis_index("s") * NC + lax.axis_index("c")
        base = wid * nb
        zero = jnp.zeros((L,), jnp.float32)

        def b_body(b, carry):
            u = [urows[b, pl.ds(L * t, L)] for t in range(4)]
            v = [vrows[b, pl.ds(L * t, L)] for t in range(4)]
            posb[b, :] = u[0] * v[0] + u[1] * v[1] + u[2] * v[2] + u[3] * v[3]

            def n_body(n, accs):
                r = b * N_NEG + n
                return tuple(accs[t] + nrows[r, pl.ds(L * t, L)] for t in range(4))

            a = lax.fori_loop(0, N_NEG, n_body, (zero, zero, zero, zero))
            negb[b, :] = a[0] * u[0] + a[1] * u[1] + a[2] * u[2] + a[3] * u[3]
            return carry

        def c_body(ci, carry):
            b0 = base + ci * CB
            cb0 = ci * CB
            hs = [pltpu.async_copy(u_hbm.at[uidx.at[pl.ds(cb0, CB)]], urows, sem),
                  pltpu.async_copy(v_hbm.at[vidx.at[pl.ds(cb0, CB)]], vrows, sem)]
            for j in range(NIDX_ROWS):
                hs.append(pltpu.async_copy(v_hbm.at[nidx.at[ci * NIDX_ROWS + j]],
                                           nrows.at[pl.ds(j * 128, 128)], sem))
            for h in hs:
                h.wait()
            carry = lax.fori_loop(0, CB, b_body, carry)
            pltpu.sync_copy(posb, pos_out.at[pl.ds(b0, CB)])
            pltpu.sync_copy(negb, neg_out.at[pl.ds(b0, CB)])
            return carry

        pltpu.sync_copy(up_hbm.at[pl.ds(base, nb)], uidx)
        pltpu.sync_copy(vp_hbm.at[pl.ds(base, nb)], vidx)
        pltpu.sync_copy(vn_hbm.at[pl.ds(wid * nidx_per_w, nidx_per_w)], nidx)
        lax.fori_loop(0, nch, c_body, 0)

    return k(U, V, u_pos, v_pos, vneg2)


def _sc_reg_kernel(U, P, u_pos, B):
    nb = B // NW
    rcb = 64
    nch = nb // rcb

    @functools.partial(
        pl.kernel,
        out_type=jax.ShapeDtypeStruct((NW, L), jnp.float32),
        mesh=plsc.VectorSubcoreMesh(**_MESH),
        scratch_types=[
            pltpu.VMEM((nb,), jnp.int32),
            pltpu.VMEM((rcb, 2 * DIM), jnp.float32),
            pltpu.VMEM((rcb, 2 * DIM), jnp.float32),
            pltpu.VMEM((L,), jnp.float32),
            pltpu.SemaphoreType.DMA,
        ],
    )
    def k(u_hbm, p_hbm, up_hbm, reg_out, uidx, urows, prows, regv, sem):
        wid = lax.axis_index("s") * NC + lax.axis_index("c")
        base = wid * nb
        zero = jnp.zeros((L,), jnp.float32)

        def b_body(b, racc):
            u = [urows[b, pl.ds(L * t, L)] for t in range(4)]
            pp = [prows[b, pl.ds(L * t, L)] for t in range(4)]
            return (racc + jnp.abs(u[0] - pp[0]) + jnp.abs(u[1] - pp[1])
                    + jnp.abs(u[2] - pp[2]) + jnp.abs(u[3] - pp[3]))

        def c_body(ci, racc):
            cb0 = ci * rcb
            hs = [pltpu.async_copy(u_hbm.at[uidx.at[pl.ds(cb0, rcb)]], urows, sem),
                  pltpu.async_copy(p_hbm.at[uidx.at[pl.ds(cb0, rcb)]], prows, sem)]
            for h in hs:
                h.wait()
            return lax.fori_loop(0, rcb, b_body, racc)

        pltpu.sync_copy(up_hbm.at[pl.ds(base, nb)], uidx)
        racc = lax.fori_loop(0, nch, c_body, zero)
        regv[...] = racc
        pltpu.sync_copy(regv, reg_out.at[wid])

    return k(U, P, u_pos)


def _tc_finalize(pos, neg, regp, B):
    def body(pos_ref, neg_ref, reg_ref, o_ref):
        s = jnp.sum(pos_ref[...], axis=1)
        t = jnp.sum(neg_ref[...], axis=1)
        ls = jnp.minimum(s, 0.0) - jnp.log1p(jnp.exp(-jnp.abs(s)))
        lt = jnp.minimum(-t, 0.0) - jnp.log1p(jnp.exp(-jnp.abs(t)))
        total = jnp.sum(ls + lt)
        reg = REG * jnp.sum(reg_ref[...])
        o_ref[...] = jnp.reshape(-(total / B) - reg, (1, 1))

    return pl.pallas_call(
        body, out_shape=jax.ShapeDtypeStruct((1, 1), jnp.float32),
    )(pos, neg, regp)


def kernel(U, V, pretrained, u_pos, v_pos, v_neg, batch_size):
    B = u_pos.shape[0]
    U1 = jnp.pad(U, ((0, 0), (0, DIM)))
    V1 = jnp.pad(V, ((0, 0), (0, DIM)))
    P1 = jnp.pad(pretrained, ((0, 0), (0, DIM)))
    up = u_pos.astype(jnp.int32)
    vp = v_pos.astype(jnp.int32)
    vn2 = v_neg.astype(jnp.int32).reshape(B * N_NEG // 128, 128)
    pos, neg = _sc_uv_kernel(U1, V1, up, vp, vn2, B)
    regp = _sc_reg_kernel(U1, P1, up, B)
    out = _tc_finalize(pos, neg, regp, B)
    return out[0, 0]


# speedup vs baseline: 1.1134x; 1.0022x over previous
import functools

import jax
import jax.numpy as jnp
from jax import lax
from jax.experimental import pallas as pl
from jax.experimental.pallas import tpu as pltpu
from jax.experimental.pallas import tpu_sc as plsc

VOCAB = 1000000
DIM = 64
REG = 1e-06
N_NEG = 20

NC = 2
NS = 16
NW = NC * NS
L = 16

CB = 32
NEG_ROWS = CB * N_NEG
NIDX_ROWS = NEG_ROWS // 128

_MESH = dict(core_axis_name="c", subcore_axis_name="s")


def _sc_uv_kernel(U, V, u_pos, v_pos, vneg2, B):
    nb = B // NW
    nch = nb // CB
    nidx_per_w = nb * N_NEG // 128

    @functools.partial(
        pl.kernel,
        out_type=(
            jax.ShapeDtypeStruct((B, L), jnp.float32),
            jax.ShapeDtypeStruct((B, L), jnp.float32),
        ),
        mesh=plsc.VectorSubcoreMesh(**_MESH),
        scratch_types=[
            pltpu.VMEM((nb,), jnp.int32),
            pltpu.VMEM((nb,), jnp.int32),
            pltpu.VMEM((nidx_per_w, 128), jnp.int32),
            pltpu.VMEM((CB, 2 * DIM), jnp.float32),
            pltpu.VMEM((CB, 2 * DIM), jnp.float32),
            pltpu.VMEM((NEG_ROWS, 2 * DIM), jnp.float32),
            pltpu.VMEM((CB, L), jnp.float32),
            pltpu.VMEM((CB, L), jnp.float32),
            pltpu.SemaphoreType.DMA,
        ],
    )
    def k(u_hbm, v_hbm, up_hbm, vp_hbm, vn_hbm,
          pos_out, neg_out,
          uidx, vidx, nidx, urows, vrows, nrows, posb, negb, sem):
        wid = lax.axis_index("s") * NC + lax.axis_index("c")
        base = wid * nb
        zero = jnp.zeros((L,), jnp.float32)

        def b_body(b, carry):
            u = [urows[b, pl.ds(L * t, L)] for t in range(4)]
            v = [vrows[b, pl.ds(L * t, L)] for t in range(4)]
            posb[b, :] = u[0] * v[0] + u[1] * v[1] + u[2] * v[2] + u[3] * v[3]

            def n_body(n, accs):
                r = b * N_NEG + n
                return tuple(accs[t] + nrows[r, pl.ds(L * t, L)] for t in range(4))

            a = lax.fori_loop(0, N_NEG, n_body, (zero, zero, zero, zero))
            negb[b, :] = a[0] * u[0] + a[1] * u[1] + a[2] * u[2] + a[3] * u[3]
            return carry

        def c_body(ci, carry):
            b0 = base + ci * CB
            cb0 = ci * CB
            hs = [pltpu.async_copy(u_hbm.at[uidx.at[pl.ds(cb0, CB)]], urows, sem),
                  pltpu.async_copy(v_hbm.at[vidx.at[pl.ds(cb0, CB)]], vrows, sem)]
            for j in range(NIDX_ROWS):
                hs.append(pltpu.async_copy(v_hbm.at[nidx.at[ci * NIDX_ROWS + j]],
                                           nrows.at[pl.ds(j * 128, 128)], sem))
            for h in hs:
                h.wait()
            carry = lax.fori_loop(0, CB, b_body, carry)
            pltpu.sync_copy(posb, pos_out.at[pl.ds(b0, CB)])
            pltpu.sync_copy(negb, neg_out.at[pl.ds(b0, CB)])
            return carry

        pltpu.sync_copy(up_hbm.at[pl.ds(base, nb)], uidx)
        pltpu.sync_copy(vp_hbm.at[pl.ds(base, nb)], vidx)
        pltpu.sync_copy(vn_hbm.at[pl.ds(wid * nidx_per_w, nidx_per_w)], nidx)
        lax.fori_loop(0, nch, c_body, 0)

    return k(U, V, u_pos, v_pos, vneg2)


def _sc_reg_kernel(U, P, u_pos, B):
    nb = B // NW
    rcb = 64
    nch = nb // rcb

    @functools.partial(
        pl.kernel,
        out_type=jax.ShapeDtypeStruct((NW, L), jnp.float32),
        mesh=plsc.VectorSubcoreMesh(**_MESH),
        scratch_types=[
            pltpu.VMEM((nb,), jnp.int32),
            pltpu.VMEM((rcb, 2 * DIM), jnp.float32),
            pltpu.VMEM((rcb, 2 * DIM), jnp.float32),
            pltpu.VMEM((rcb, 2 * DIM), jnp.float32),
            pltpu.VMEM((rcb, 2 * DIM), jnp.float32),
            pltpu.VMEM((L,), jnp.float32),
            pltpu.SemaphoreType.DMA,
            pltpu.SemaphoreType.DMA,
        ],
    )
    def k(u_hbm, p_hbm, up_hbm, reg_out,
          uidx, urows0, prows0, urows1, prows1, regv, sem0, sem1):
        wid = lax.axis_index("s") * NC + lax.axis_index("c")
        base = wid * nb
        zero = jnp.zeros((L,), jnp.float32)
        bufs = ((urows0, prows0, sem0), (urows1, prows1, sem1))

        def issue(ci, s):
            urows, prows, sem = bufs[s]
            cb0 = ci * rcb
            pltpu.async_copy(u_hbm.at[uidx.at[pl.ds(cb0, rcb)]], urows, sem)
            pltpu.async_copy(p_hbm.at[uidx.at[pl.ds(cb0, rcb)]], prows, sem)

        def drain(ci, s):
            urows, prows, sem = bufs[s]
            cb0 = ci * rcb
            pltpu.make_async_copy(u_hbm.at[uidx.at[pl.ds(cb0, rcb)]], urows, sem).wait()
            pltpu.make_async_copy(p_hbm.at[uidx.at[pl.ds(cb0, rcb)]], prows, sem).wait()

        def compute(s, racc):
            urows, prows, _ = bufs[s]

            def b_body(b, racc):
                u = [urows[b, pl.ds(L * t, L)] for t in range(4)]
                pp = [prows[b, pl.ds(L * t, L)] for t in range(4)]
                return (racc + jnp.abs(u[0] - pp[0]) + jnp.abs(u[1] - pp[1])
                        + jnp.abs(u[2] - pp[2]) + jnp.abs(u[3] - pp[3]))

            return lax.fori_loop(0, rcb, b_body, racc)

        pltpu.sync_copy(up_hbm.at[pl.ds(base, nb)], uidx)
        issue(0, 0)

        def pair_body(cj, racc):
            ci = cj * 2
            issue(ci + 1, 1)
            drain(ci, 0)
            racc = compute(0, racc)
            nxt = jnp.minimum(ci + 2, nch - 1)
            issue(nxt, 0)
            drain(ci + 1, 1)
            return compute(1, racc)

        racc = lax.fori_loop(0, nch // 2, pair_body, zero)
        pltpu.make_async_copy(u_hbm.at[uidx.at[pl.ds(0, rcb)]], urows0, sem0).wait()
        pltpu.make_async_copy(p_hbm.at[uidx.at[pl.ds(0, rcb)]], prows0, sem0).wait()
        regv[...] = racc
        pltpu.sync_copy(regv, reg_out.at[wid])

    return k(U, P, u_pos)


def _tc_finalize(pos, neg, regp, B):
    def body(pos_ref, neg_ref, reg_ref, o_ref):
        s = jnp.sum(pos_ref[...], axis=1)
        t = jnp.sum(neg_ref[...], axis=1)
        ls = jnp.minimum(s, 0.0) - jnp.log1p(jnp.exp(-jnp.abs(s)))
        lt = jnp.minimum(-t, 0.0) - jnp.log1p(jnp.exp(-jnp.abs(t)))
        total = jnp.sum(ls + lt)
        reg = REG * jnp.sum(reg_ref[...])
        o_ref[...] = jnp.reshape(-(total / B) - reg, (1, 1))

    return pl.pallas_call(
        body, out_shape=jax.ShapeDtypeStruct((1, 1), jnp.float32),
    )(pos, neg, regp)


def kernel(U, V, pretrained, u_pos, v_pos, v_neg, batch_size):
    B = u_pos.shape[0]
    U1 = jnp.pad(U, ((0, 0), (0, DIM)))
    V1 = jnp.pad(V, ((0, 0), (0, DIM)))
    P1 = jnp.pad(pretrained, ((0, 0), (0, DIM)))
    up = u_pos.astype(jnp.int32)
    vp = v_pos.astype(jnp.int32)
    vn2 = v_neg.astype(jnp.int32).reshape(B * N_NEG // 128, 128)
    pos, neg = _sc_uv_kernel(U1, V1, up, vp, vn2, B)
    regp = _sc_reg_kernel(U1, P1, up, B)
    out = _tc_finalize(pos, neg, regp, B)
    return out[0, 0]
